# baseline (speedup 1.0000x reference)
import numpy as np
import jax
import jax.numpy as jnp
from jax import lax
from jax.experimental import pallas as pl
from jax.experimental.pallas import tpu as pltpu

_EPS = 1e-5
_NCLS = 19.0
_K = 6
_CC = 64


def _resize_wmat(in_size: int, out_size: int) -> np.ndarray:
    scale = out_size / in_size
    inv_scale = 1.0 / scale
    kernel_scale = max(inv_scale, 1.0)
    sample_f = (np.arange(out_size, dtype=np.float64) + 0.5) * inv_scale - 0.5
    x = np.abs(sample_f[None, :] - np.arange(in_size, dtype=np.float64)[:, None]) / kernel_scale
    w = np.maximum(0.0, 1.0 - x)
    tot = w.sum(axis=0, keepdims=True)
    w = np.where(np.abs(tot) > 1000 * np.finfo(np.float32).eps,
                 w / np.where(tot != 0.0, tot, 1.0), 0.0)
    w = np.where(((sample_f >= -0.5) & (sample_f <= in_size - 0.5))[None, :], w, 0.0)
    return w.astype(np.float32)


def _flat_to_2d(v, H):
    return jnp.concatenate([v[:, h * H:(h + 1) * H] for h in range(H)], axis=0)


def _2d_to_flat(a, H):
    return jnp.concatenate([a[h:h + 1, :] for h in range(H)], axis=1)


def _conv_bands(conv_w, H):
    D = np.zeros((7, H, H), dtype=np.float32)
    for dw in range(7):
        for w in range(H):
            wp = w + dw - 3
            if 0 <= wp < H:
                D[dw, wp, w] = 1.0
    b = jnp.einsum('cdk,kij->dcij', conv_w[0], jnp.asarray(D),
                   precision=lax.Precision.HIGHEST)
    return b.reshape(7, 3 * H, H)


def _in_copy(x_hbm, xv, in_sem, batch, slot, c):
    return pltpu.make_async_copy(
        x_hbm.at[batch, pl.ds(c * _CC, _CC), :],
        xv.at[slot, pl.ds(c * _CC, _CC), :],
        in_sem.at[slot, c],
    )


def _out_copy(out_hbm, ov, out_sem, batch, slot, c):
    return pltpu.make_async_copy(
        ov.at[slot, pl.ds(c * _CC, _CC), :],
        out_hbm.at[batch, pl.ds(c * _CC, _CC), :],
        out_sem.at[slot],
    )


def _san_body(mask_ref, wt_ref, w_ref, g_ref, b_ref, bands_ref, x_hbm, out_hbm,
              xv, ov, in_sem, out_sem):
    f32 = jnp.float32
    hi = lax.Precision.HIGHEST
    C = xv.shape[1]
    HW = xv.shape[2]
    H = 56
    n = pl.num_programs(0)
    i = pl.program_id(0)
    slot = lax.rem(i, 2)
    nslot = lax.rem(i + 1, 2)

    @pl.when(i == 0)
    def _prologue():
        for c in range(_K):
            _in_copy(x_hbm, xv, in_sem, 0, 0, c).start()

    @pl.when(i < n - 1)
    def _prefetch_next():
        for c in range(_K):
            _in_copy(x_hbm, xv, in_sem, i + 1, nslot, c).start()

    @pl.when(i >= 2)
    def _drain_old_out():
        for c in range(_K):
            _out_copy(out_hbm, ov, out_sem, i - 2, slot, c).wait()

    mask = mask_ref[0]
    t = jnp.dot(wt_ref[...], mask, preferred_element_type=f32, precision=hi)
    m2 = jnp.dot(t, w_ref[...], preferred_element_type=f32, precision=hi)

    ones = jnp.full((1, _CC), 1.0 / C, f32)
    xsum = jnp.zeros((1, HW), f32)
    xmax = jnp.full((1, HW), -jnp.inf, f32)
    for c in range(_K):
        _in_copy(x_hbm, xv, in_sem, i, slot, c).wait()
        xc = xv[slot, c * _CC:(c + 1) * _CC]
        xsum = xsum + jnp.dot(ones, xc, preferred_element_type=f32, precision=hi)
        xmax = jnp.maximum(xmax, jnp.max(xc, axis=0, keepdims=True))
    avg2 = _flat_to_2d(xsum, H) * m2
    mx2 = _flat_to_2d(xmax, H) * m2

    iconcat = jnp.concatenate([avg2, mx2, m2], axis=1)
    acc = jnp.zeros((H, H), f32)
    for dh in range(7):
        md = jnp.dot(iconcat, bands_ref[dh],
                     preferred_element_type=f32, precision=hi)
        s = dh - 3
        if s > 0:
            md = jnp.concatenate([md[s:, :], jnp.zeros((s, H), f32)], axis=0)
        elif s < 0:
            md = jnp.concatenate([jnp.zeros((-s, H), f32), md[:H + s, :]], axis=0)
        acc = acc + md
    atten = 1.0 / (1.0 + jnp.exp(-acc))
    matten = _2d_to_flat(m2 * atten, H)

    for c in range(_K):
        sl = slice(c * _CC, (c + 1) * _CC)
        o = xv[slot, sl] * matten
        s1 = jnp.sum(o, axis=1, keepdims=True)
        s2 = jnp.sum(o * o, axis=1, keepdims=True)
        mean = s1 * (1.0 / HW)
        var = s2 * (1.0 / HW) - mean * mean
        scale = lax.rsqrt(var + _EPS) * g_ref[sl, :]
        bias = b_ref[sl, :] - mean * scale
        ov[slot, sl] = jnp.maximum(o * scale + bias, 0.0)
        _out_copy(out_hbm, ov, out_sem, i, slot, c).start()

    @pl.when(i == n - 1)
    def _epilogue():
        for c in range(_K):
            _out_copy(out_hbm, ov, out_sem, i - 1, nslot, c).wait()
        for c in range(_K):
            _out_copy(out_hbm, ov, out_sem, i, slot, c).wait()


def kernel(x, masks, conv_w, gamma, beta):
    n, C, H, W = x.shape
    HW = H * W
    wmat = _resize_wmat(masks.shape[-1], H)
    wt = jnp.asarray(wmat.T)
    wm = jnp.asarray(wmat)
    g19 = (gamma.astype(jnp.float32) * _NCLS).reshape(C, 1)
    b19 = (beta.astype(jnp.float32) * _NCLS).reshape(C, 1)
    xf = x.reshape(n, C, HW)

    out = pl.pallas_call(
        _san_body,
        grid=(n,),
        in_specs=[
            pl.BlockSpec((1, masks.shape[1], masks.shape[2]), lambda i: (i, 0, 0)),
            pl.BlockSpec((H, masks.shape[1]), lambda i: (0, 0)),
            pl.BlockSpec((masks.shape[1], H), lambda i: (0, 0)),
            pl.BlockSpec((C, 1), lambda i: (0, 0)),
            pl.BlockSpec((C, 1), lambda i: (0, 0)),
            pl.BlockSpec((7, 3 * H, H), lambda i: (0, 0, 0)),
            pl.BlockSpec(memory_space=pltpu.MemorySpace.HBM),
        ],
        out_specs=pl.BlockSpec(memory_space=pltpu.MemorySpace.HBM),
        out_shape=jax.ShapeDtypeStruct((n, C, HW), x.dtype),
        scratch_shapes=[
            pltpu.VMEM((2, C, HW), jnp.float32),
            pltpu.VMEM((2, C, HW), jnp.float32),
            pltpu.SemaphoreType.DMA((2, _K)),
            pltpu.SemaphoreType.DMA((2,)),
        ],
        compiler_params=pltpu.CompilerParams(
            dimension_semantics=("arbitrary",),
        ),
    )(masks, wt, wm, g19, b19, _conv_bands(conv_w, H), xf)
    return out.reshape(n, C, H, W)


# speedup vs baseline: 1.0576x; 1.0576x over previous
import numpy as np
import jax
import jax.numpy as jnp
from jax import lax
from jax.experimental import pallas as pl
from jax.experimental.pallas import tpu as pltpu

_EPS = 1e-5
_NCLS = 19.0
_CHUNK = 64


def _resize_wmat(in_size: int, out_size: int) -> np.ndarray:
    scale = out_size / in_size
    inv_scale = 1.0 / scale
    kernel_scale = max(inv_scale, 1.0)
    sample_f = (np.arange(out_size, dtype=np.float64) + 0.5) * inv_scale - 0.5
    x = np.abs(sample_f[None, :] - np.arange(in_size, dtype=np.float64)[:, None]) / kernel_scale
    w = np.maximum(0.0, 1.0 - x)
    tot = w.sum(axis=0, keepdims=True)
    w = np.where(np.abs(tot) > 1000 * np.finfo(np.float32).eps,
                 w / np.where(tot != 0.0, tot, 1.0), 0.0)
    w = np.where(((sample_f >= -0.5) & (sample_f <= in_size - 0.5))[None, :], w, 0.0)
    return w.astype(np.float32)


def _flat_to_2d(v, H):
    return jnp.concatenate([v[:, h * H:(h + 1) * H] for h in range(H)], axis=0)


def _2d_to_flat(a, H):
    return jnp.concatenate([a[h:h + 1, :] for h in range(H)], axis=1)


def _conv_bands(conv_w, H):
    D = np.zeros((7, H, H), dtype=np.float32)
    for dw in range(7):
        for w in range(H):
            wp = w + dw - 3
            if 0 <= wp < H:
                D[dw, wp, w] = 1.0
    b = jnp.einsum('cdk,kij->dcij', conv_w[0], jnp.asarray(D),
                   precision=lax.Precision.HIGHEST)
    return b.reshape(7, 3 * H, H)


def _san_body(mask_ref, x_ref, wt_ref, w_ref, g_ref, b_ref, bands_ref, out_ref):
    f32 = jnp.float32
    hi = lax.Precision.HIGHEST
    C = x_ref.shape[1]
    HW = x_ref.shape[2]
    H = 56

    mask = mask_ref[0]
    t = jnp.dot(wt_ref[...], mask, preferred_element_type=f32, precision=hi)
    m2 = jnp.dot(t, w_ref[...], preferred_element_type=f32, precision=hi)

    xall = x_ref[0]
    ones = jnp.full((1, C), 1.0 / C, f32)
    xmean = jnp.dot(ones, xall, preferred_element_type=f32, precision=hi)
    xmax = jnp.max(xall, axis=0, keepdims=True)
    avg2 = _flat_to_2d(xmean, H) * m2
    mx2 = _flat_to_2d(xmax, H) * m2

    iconcat = jnp.concatenate([avg2, mx2, m2], axis=1)
    acc = jnp.zeros((H, H), f32)
    for dh in range(7):
        md = jnp.dot(iconcat, bands_ref[dh], preferred_element_type=f32)
        s = dh - 3
        if s > 0:
            md = jnp.concatenate([md[s:, :], jnp.zeros((s, H), f32)], axis=0)
        elif s < 0:
            md = jnp.concatenate([jnp.zeros((-s, H), f32), md[:H + s, :]], axis=0)
        acc = acc + md
    atten = 1.0 / (1.0 + jnp.exp(-acc))
    matten = _2d_to_flat(m2 * atten, H)

    for i in range(C // _CHUNK):
        sl = slice(i * _CHUNK, (i + 1) * _CHUNK)
        o = x_ref[0, sl] * matten
        s1 = jnp.sum(o, axis=1, keepdims=True)
        s2 = jnp.sum(o * o, axis=1, keepdims=True)
        mean = s1 * (1.0 / HW)
        var = s2 * (1.0 / HW) - mean * mean
        scale = lax.rsqrt(var + _EPS) * g_ref[sl, :]
        bias = b_ref[sl, :] - mean * scale
        out_ref[0, sl] = jnp.maximum(o * scale + bias, 0.0)


def kernel(x, masks, conv_w, gamma, beta):
    n, C, H, W = x.shape
    HW = H * W
    wmat = _resize_wmat(masks.shape[-1], H)
    wt = jnp.asarray(wmat.T)
    wm = jnp.asarray(wmat)
    g19 = (gamma.astype(jnp.float32) * _NCLS).reshape(C, 1)
    b19 = (beta.astype(jnp.float32) * _NCLS).reshape(C, 1)
    xf = x.reshape(n, C, HW)

    out = pl.pallas_call(
        _san_body,
        grid=(n,),
        in_specs=[
            pl.BlockSpec((1, masks.shape[1], masks.shape[2]), lambda i: (i, 0, 0)),
            pl.BlockSpec((1, C, HW), lambda i: (i, 0, 0)),
            pl.BlockSpec((H, masks.shape[1]), lambda i: (0, 0)),
            pl.BlockSpec((masks.shape[1], H), lambda i: (0, 0)),
            pl.BlockSpec((C, 1), lambda i: (0, 0)),
            pl.BlockSpec((C, 1), lambda i: (0, 0)),
            pl.BlockSpec((7, 3 * H, H), lambda i: (0, 0, 0)),
        ],
        out_specs=pl.BlockSpec((1, C, HW), lambda i: (i, 0, 0)),
        out_shape=jax.ShapeDtypeStruct((n, C, HW), x.dtype),
        compiler_params=pltpu.CompilerParams(
            dimension_semantics=("arbitrary",),
        ),
    )(masks, xf, wt, wm, g19, b19, _conv_bands(conv_w, H))
    return out.reshape(n, C, H, W)


# speedup vs baseline: 1.1534x; 1.0905x over previous
import numpy as np
import jax
import jax.numpy as jnp
from jax import lax
from jax.experimental import pallas as pl
from jax.experimental.pallas import tpu as pltpu

_EPS = 1e-5
_NCLS = 19.0
_CHUNK = 64


def _resize_wmat(in_size: int, out_size: int) -> np.ndarray:
    scale = out_size / in_size
    inv_scale = 1.0 / scale
    kernel_scale = max(inv_scale, 1.0)
    sample_f = (np.arange(out_size, dtype=np.float64) + 0.5) * inv_scale - 0.5
    x = np.abs(sample_f[None, :] - np.arange(in_size, dtype=np.float64)[:, None]) / kernel_scale
    w = np.maximum(0.0, 1.0 - x)
    tot = w.sum(axis=0, keepdims=True)
    w = np.where(np.abs(tot) > 1000 * np.finfo(np.float32).eps,
                 w / np.where(tot != 0.0, tot, 1.0), 0.0)
    w = np.where(((sample_f >= -0.5) & (sample_f <= in_size - 0.5))[None, :], w, 0.0)
    return w.astype(np.float32)


def _flat_to_2d(v, H):
    return jnp.concatenate([v[:, h * H:(h + 1) * H] for h in range(H)], axis=0)


def _2d_to_flat(a, H):
    return jnp.concatenate([a[h:h + 1, :] for h in range(H)], axis=1)


def _conv_bands(conv_w, H):
    D = np.zeros((7, H, H), dtype=np.float32)
    for dw in range(7):
        for w in range(H):
            wp = w + dw - 3
            if 0 <= wp < H:
                D[dw, wp, w] = 1.0
    b = jnp.einsum('cdk,kij->dcij', conv_w[0], jnp.asarray(D),
                   precision=lax.Precision.HIGHEST)
    return b.reshape(7, 3 * H, H)


def _san_body(mask_ref, x_ref, wt_ref, w_ref, g_ref, b_ref, bands_ref, out_ref):
    f32 = jnp.float32
    C = x_ref.shape[1]
    HW = x_ref.shape[2]
    H = 56

    mask = mask_ref[0]
    t = jnp.dot(wt_ref[...], mask, preferred_element_type=f32)
    m2 = jnp.dot(t, w_ref[...], preferred_element_type=f32)

    xall = x_ref[0]
    ones = jnp.full((1, C), 1.0 / C, f32)
    xmean = jnp.dot(ones, xall, preferred_element_type=f32)
    xmax = jnp.max(xall, axis=0, keepdims=True)
    avg2 = _flat_to_2d(xmean, H) * m2
    mx2 = _flat_to_2d(xmax, H) * m2

    iconcat = jnp.concatenate([avg2, mx2, m2], axis=1)
    acc = jnp.zeros((H, H), f32)
    for dh in range(7):
        md = jnp.dot(iconcat, bands_ref[dh], preferred_element_type=f32)
        s = dh - 3
        if s > 0:
            md = jnp.concatenate([md[s:, :], jnp.zeros((s, H), f32)], axis=0)
        elif s < 0:
            md = jnp.concatenate([jnp.zeros((-s, H), f32), md[:H + s, :]], axis=0)
        acc = acc + md
    atten = 1.0 / (1.0 + jnp.exp(-acc))
    matten = _2d_to_flat(m2 * atten, H)

    for i in range(C // _CHUNK):
        sl = slice(i * _CHUNK, (i + 1) * _CHUNK)
        o = x_ref[0, sl] * matten
        s1 = jnp.sum(o, axis=1, keepdims=True)
        s2 = jnp.sum(o * o, axis=1, keepdims=True)
        mean = s1 * (1.0 / HW)
        var = s2 * (1.0 / HW) - mean * mean
        scale = lax.rsqrt(var + _EPS) * g_ref[sl, :]
        bias = b_ref[sl, :] - mean * scale
        out_ref[0, sl] = jnp.maximum(o * scale + bias, 0.0)


def kernel(x, masks, conv_w, gamma, beta):
    n, C, H, W = x.shape
    HW = H * W
    wmat = _resize_wmat(masks.shape[-1], H)
    wt = jnp.asarray(wmat.T)
    wm = jnp.asarray(wmat)
    g19 = (gamma.astype(jnp.float32) * _NCLS).reshape(C, 1)
    b19 = (beta.astype(jnp.float32) * _NCLS).reshape(C, 1)
    xf = x.reshape(n, C, HW)

    out = pl.pallas_call(
        _san_body,
        grid=(n,),
        in_specs=[
            pl.BlockSpec((1, masks.shape[1], masks.shape[2]), lambda i: (i, 0, 0)),
            pl.BlockSpec((1, C, HW), lambda i: (i, 0, 0)),
            pl.BlockSpec((H, masks.shape[1]), lambda i: (0, 0)),
            pl.BlockSpec((masks.shape[1], H), lambda i: (0, 0)),
            pl.BlockSpec((C, 1), lambda i: (0, 0)),
            pl.BlockSpec((C, 1), lambda i: (0, 0)),
            pl.BlockSpec((7, 3 * H, H), lambda i: (0, 0, 0)),
        ],
        out_specs=pl.BlockSpec((1, C, HW), lambda i: (i, 0, 0)),
        out_shape=jax.ShapeDtypeStruct((n, C, HW), x.dtype),
        compiler_params=pltpu.CompilerParams(
            dimension_semantics=("arbitrary",),
        ),
    )(masks, xf, wt, wm, g19, b19, _conv_bands(conv_w, H))
    return out.reshape(n, C, H, W)
